# Initial kernel scaffold; baseline (speedup 1.0000x reference)
import jax
import jax.numpy as jnp
from jax.experimental import pallas as pl


def kernel(ids, bigram_weight, tri_weight):
    raise NotImplementedError("write your pallas kernel here")


# speedup vs baseline: 1.2530x; 1.2530x over previous
import functools

import jax
import jax.numpy as jnp
from jax import lax
from jax.experimental import pallas as pl
from jax.experimental.pallas import tpu as pltpu, tpu_sc as plsc

VOCAB = 1000000
DIM = 32
ROWS = 4096
COLS = 200
PCOLS = 224
NCHUNK = PCOLS // 16
SLAB = 112
NW = 32
ROWS_PER_W = ROWS // NW


def _body(ids_hbm, bg_hbm, tri_hbm, out_hbm, ids_v, idx_bi0, idx_bi1,
          idx_tri0, idx_tri1, out_v, tri_v, sem):
    wid = lax.axis_index("s") * 2 + lax.axis_index("c")
    r0 = wid * ROWS_PER_W

    pltpu.sync_copy(ids_hbm.at[pl.ds(r0 * COLS, ROWS_PER_W * COLS)], ids_v)

    @pl.loop(0, ROWS_PER_W)
    def _row(r):
        base = jnp.full((16,), r * COLS, dtype=jnp.int32)
        lane = lax.iota(jnp.int32, 16)
        for j in range(NCHUNK):
            col = lane + (j * 16)
            cc = jnp.minimum(col, COLS - 1)
            cur = plsc.load_gather(ids_v, [base + cc])
            pc = jnp.minimum(jnp.maximum(col - 1, 0), COLS - 1)
            prev = plsc.load_gather(ids_v, [base + pc])
            p2c = jnp.minimum(jnp.maximum(col - 2, 0), COLS - 1)
            prev2 = plsc.load_gather(ids_v, [base + p2c])
            if j == 0:
                zero = jnp.zeros((16,), jnp.int32)
                prev = jnp.where(col >= 1, prev, zero)
                prev2 = jnp.where(col >= 2, prev2, zero)
            t = prev * 131 + cur
            bi = lax.rem(t, VOCAB)
            tri = lax.rem(prev2 * 173 + t, VOCAB)
            c, off = divmod(j * 16, SLAB)
            ib = idx_bi0 if c == 0 else idx_bi1
            it = idx_tri0 if c == 0 else idx_tri1
            ib[pl.ds(off, 16)] = bi
            it[pl.ds(off, 16)] = tri

        cps = [
            pltpu.async_copy(bg_hbm.at[idx_bi0],
                             out_v.at[pl.ds(0, SLAB)], sem),
            pltpu.async_copy(bg_hbm.at[idx_bi1],
                             out_v.at[pl.ds(SLAB, SLAB)], sem),
            pltpu.async_copy(tri_hbm.at[idx_tri0],
                             tri_v.at[pl.ds(0, SLAB)], sem),
            pltpu.async_copy(tri_hbm.at[idx_tri1],
                             tri_v.at[pl.ds(SLAB, SLAB)], sem),
        ]
        for cp in cps:
            cp.wait()

        @pl.loop(0, COLS, unroll=4)
        def _acc(k):
            trow = tri_v.at[k]
            orow = out_v.at[k]
            for h in range(2):
                orow[pl.ds(h * 16, 16)] += trow[pl.ds(h * 16, 16)]

        pltpu.sync_copy(out_v.at[pl.ds(0, COLS)], out_hbm.at[r0 + r])


@functools.partial(jax.jit, donate_argnums=())
def _run(ids_flat, bg, tw):
    mesh = plsc.VectorSubcoreMesh(core_axis_name="c", subcore_axis_name="s")
    return pl.kernel(
        _body,
        out_type=jax.ShapeDtypeStruct((ROWS, COLS, DIM), jnp.float32),
        mesh=mesh,
        compiler_params=pltpu.CompilerParams(
            needs_layout_passes=False, use_tc_tiling_on_sc=False),
        scratch_types=[
            pltpu.VMEM((ROWS_PER_W * COLS,), jnp.int32),
            pltpu.VMEM((SLAB,), jnp.int32),
            pltpu.VMEM((SLAB,), jnp.int32),
            pltpu.VMEM((SLAB,), jnp.int32),
            pltpu.VMEM((SLAB,), jnp.int32),
            pltpu.VMEM((PCOLS, DIM), jnp.float32),
            pltpu.VMEM((PCOLS, DIM), jnp.float32),
            pltpu.SemaphoreType.DMA,
        ],
    )(ids_flat, bg, tw)


def kernel(ids, bigram_weight, tri_weight):
    return _run(ids.astype(jnp.int32).reshape(-1), bigram_weight, tri_weight)
